# baseline (speedup 1.0000x reference)
import functools

import jax
import jax.numpy as jnp
from jax import lax
from jax.experimental import pallas as pl
from jax.experimental.pallas import tpu as pltpu
from jax.experimental.pallas import tpu_sc as plsc

BATCH = 524288
DIM = 32
LABELS = 8192

SC_N = 32768
B0 = BATCH - SC_N
NW = 32
SC_W = SC_N
T0 = B0 // 128
TPW = SC_W // 1024
TPC = 32
NCHUNK = TPW // TPC
TILES = TPC

BLKC = 32768
GRID = B0 // BLKC
ACCW = 1024
SLICES = BLKC // ACCW

_DN_LANE = (((1,), (0,)), ((), ()))
_DN_LAST = (((1,), (1,)), ((), ()))

_mesh = plsc.VectorSubcoreMesh(core_axis_name="c", subcore_axis_name="s")


@functools.partial(
    pl.kernel,
    mesh=_mesh,
    out_type=jax.ShapeDtypeStruct((NW * 128,), jnp.float32),
    scratch_types=[
        pltpu.VMEM((TPC, 8, 128), jnp.float32),
        pltpu.VMEM((TPC, 8, 128), jnp.float32),
        pltpu.VMEM((128,), jnp.float32),
        pltpu.SemaphoreType.DMA,
        pltpu.SemaphoreType.DMA,
    ],
)
def _colsum_sc(x_hbm, out_hbm, buf0, buf1, stage, sem0, sem1):
    wid = lax.axis_index("s") * 2 + lax.axis_index("c")
    g = wid // 8
    j = wid % 8
    t0 = T0 + j * TPW
    bufs = (buf0, buf1)
    sems = (sem0, sem1)
    copies = [
        pltpu.async_copy(x_hbm.at[g, pl.ds(t0 + c * TPC, TPC)], bufs[c], sems[c])
        for c in range(min(2, NCHUNK))
    ]
    zero = jnp.zeros((16,), jnp.float32)
    accs = (zero,) * 8
    for c in range(NCHUNK):
        b = c % 2
        copies[b].wait()
        buf = bufs[b]

        def body(t, a, buf=buf):
            new = list(a)
            for k in range(8):
                for v in range(8):
                    new[k] = new[k] + buf[t, k, pl.ds(v * 16, 16)]
            return tuple(new)

        accs = lax.fori_loop(0, TILES, body, accs)
        nxt = c + 2
        if nxt < NCHUNK:
            copies[b] = pltpu.async_copy(
                x_hbm.at[g, pl.ds(t0 + nxt * TPC, TPC)], bufs[b], sems[b]
            )
    for k in range(8):
        stage[pl.ds(16 * k, 16)] = accs[k]
    pltpu.sync_copy(stage, out_hbm.at[pl.ds(wid * 128, 128)])


def _reduce_tc(x_ref, o_ref, acc_ref):
    i = pl.program_id(0)

    @pl.when(i == 0)
    def _():
        acc_ref[...] = jnp.zeros_like(acc_ref)

    a = acc_ref[...]
    x = x_ref[...]
    for k in range(SLICES):
        a += x[:, k * ACCW:(k + 1) * ACCW]
    acc_ref[...] = a

    @pl.when(i == GRID - 1)
    def _():
        acc = acc_ref[...]
        sw = lax.dot_general(
            acc, jnp.ones((ACCW, 8), jnp.float32), _DN_LANE,
            preferred_element_type=jnp.float32,
        )
        o_ref[...] = 0.125 * lax.dot_general(
            jnp.ones((8, 8), jnp.float32), sw, _DN_LAST,
            preferred_element_type=jnp.float32,
        )


def _codebook_tc(s_ref, p_ref, y_ref, o_ref):
    p = p_ref[...]
    i0 = lax.broadcasted_iota(jnp.int32, (128, 8), 0) // 16
    i1 = lax.broadcasted_iota(jnp.int32, (128, 8), 1)
    fold = jnp.where(i0 == i1, 1.0, 0.0).astype(jnp.float32)
    r = lax.dot_general(p, fold, _DN_LANE,
                        preferred_element_type=jnp.float32)
    ones8 = jnp.ones((1, 8), jnp.float32)
    parts = [
        lax.dot_general(ones8, r[8 * g:8 * g + 8, :], _DN_LANE,
                        preferred_element_type=jnp.float32)
        for g in range(4)
    ]
    s_sc = jnp.concatenate(parts, axis=1)
    s = s_ref[0:1, :] + s_sc
    sb = jnp.broadcast_to(s, (8, DIM))
    yt = y_ref[...]
    m8 = lax.dot_general(sb, yt, _DN_LANE, preferred_element_type=jnp.float32)
    q8 = lax.dot_general(
        jnp.ones((8, DIM), jnp.float32), yt * yt, _DN_LANE,
        preferred_element_type=jnp.float32,
    )
    m = m8[0:1, :]
    q = q8[0:1, :]
    metric = jnp.sign(m) * (m * m) / q
    maxv = jnp.max(metric)
    col = lax.broadcasted_iota(jnp.int32, metric.shape, 1)
    cand = jnp.where(metric == maxv, col, 2**30)
    o_ref[0, 0] = jnp.min(cand)


def kernel(inputs, mean_distances):
    xt = inputs.T
    x4 = xt.reshape(4, 8, 4096, 128).transpose(0, 2, 1, 3)
    yt = mean_distances.T

    psc = _colsum_sc(x4)

    s_tc = pl.pallas_call(
        _reduce_tc,
        grid=(GRID,),
        in_specs=[pl.BlockSpec((DIM, BLKC), lambda i: (0, i))],
        out_specs=pl.BlockSpec((8, DIM), lambda i: (0, 0)),
        out_shape=jax.ShapeDtypeStruct((8, DIM), jnp.float32),
        scratch_shapes=[pltpu.VMEM((DIM, ACCW), jnp.float32)],
    )(xt)

    idx = pl.pallas_call(
        _codebook_tc,
        out_specs=pl.BlockSpec(memory_space=pltpu.SMEM),
        out_shape=jax.ShapeDtypeStruct((1, 1), jnp.int32),
    )(s_tc, psc.reshape(NW, 128), yt)
    return idx.reshape(1)


# speedup vs baseline: 1.8657x; 1.8657x over previous
import jax
import jax.numpy as jnp
from jax import lax
from jax.experimental import pallas as pl
from jax.experimental.pallas import tpu as pltpu

BATCH = 524288
DIM = 32
LABELS = 8192
BLKC = 65536
GRID = BATCH // BLKC
ACCW = 1024
SLICES = BLKC // ACCW

_DN_LANE = (((1,), (0,)), ((), ()))
_DN_LAST = (((1,), (1,)), ((), ()))


def _fused_tc(x_ref, y_ref, o_ref, acc_ref):
    i = pl.program_id(0)

    @pl.when(i == 0)
    def _():
        acc_ref[...] = jnp.zeros_like(acc_ref)

    a = acc_ref[...]
    x = x_ref[...]
    for k in range(SLICES):
        a += x[:, k * ACCW:(k + 1) * ACCW]
    acc_ref[...] = a

    @pl.when(i == GRID - 1)
    def _():
        acc = acc_ref[...]
        sw = lax.dot_general(
            acc, jnp.ones((ACCW, 8), jnp.float32), _DN_LANE,
            preferred_element_type=jnp.float32,
        )
        s8 = 0.125 * lax.dot_general(
            jnp.ones((8, 8), jnp.float32), sw, _DN_LAST,
            preferred_element_type=jnp.float32,
        )
        yt = y_ref[...]
        m8 = lax.dot_general(s8, yt, _DN_LANE, preferred_element_type=jnp.float32)
        q8 = lax.dot_general(
            jnp.ones((8, DIM), jnp.float32), yt * yt, _DN_LANE,
            preferred_element_type=jnp.float32,
        )
        m = m8[0:1, :]
        q = q8[0:1, :]
        metric = jnp.sign(m) * (m * m) / q
        maxv = jnp.max(metric)
        col = lax.broadcasted_iota(jnp.int32, metric.shape, 1)
        cand = jnp.where(metric == maxv, col, 2**30)
        o_ref[0, 0] = jnp.min(cand)


def kernel(inputs, mean_distances):
    xt = inputs.T
    yt = mean_distances.T
    idx = pl.pallas_call(
        _fused_tc,
        grid=(GRID,),
        in_specs=[
            pl.BlockSpec((DIM, BLKC), lambda i: (0, i)),
            pl.BlockSpec((DIM, LABELS), lambda i: (0, 0)),
        ],
        out_specs=pl.BlockSpec(memory_space=pltpu.SMEM),
        out_shape=jax.ShapeDtypeStruct((1, 1), jnp.int32),
        scratch_shapes=[pltpu.VMEM((DIM, ACCW), jnp.float32)],
    )(xt, yt)
    return idx.reshape(1)
